# baseline (speedup 1.0000x reference)
import functools

import jax
import jax.numpy as jnp
import numpy as np
from jax import lax
from jax.experimental import pallas as pl
from jax.experimental.pallas import tpu as pltpu
from jax.experimental.pallas import tpu_sc as plsc

DICT = 1000000
BATCH = 4096
SEQ = 50
DIM = 64
LANES = 16

NUM_CORES = 2
NUM_SUBCORES = 16
NUM_WORKERS = NUM_CORES * NUM_SUBCORES

ITEMS_PER_WORKER = BATCH // NUM_WORKERS
CHUNK_ITEMS = 2
CHUNK_ROWS = CHUNK_ITEMS * SEQ
NUM_CHUNKS = ITEMS_PER_WORKER // CHUNK_ITEMS
NBUF = 2
NSTEPS = NUM_CHUNKS // NBUF

PACK_SPLIT = 262144
PACK_COLS = 4096
PACK_SUB = 512
PACK_GRID = PACK_SPLIT // PACK_COLS

_mesh = plsc.VectorSubcoreMesh(
    core_axis_name="c", subcore_axis_name="s",
    num_cores=NUM_CORES, num_subcores=NUM_SUBCORES)


def _pack_tc(a_ref, b_ref, c_ref, d_ref, o_ref):
    for q, ref in enumerate((a_ref, b_ref, c_ref, d_ref)):
        for k in range(PACK_COLS // PACK_SUB):
            s = k * PACK_SUB
            y = jnp.transpose(ref[:, pl.ds(s, PACK_SUB)])
            u = jax.lax.bitcast_convert_type(y, jnp.uint32)
            u = u + 0x7FFF + ((u >> 16) & 1)
            packed = (u[:, :DIM // 2] >> 16) | (u[:, DIM // 2:] & jnp.uint32(0xFFFF0000))
            o_ref[pl.ds(s, PACK_SUB), pl.ds(q * 32, 32)] = (
                jax.lax.bitcast_convert_type(packed, jnp.int32))


_pack_call = pl.pallas_call(
    _pack_tc,
    out_shape=jax.ShapeDtypeStruct((PACK_SPLIT, 2 * DIM), jnp.int32),
    grid=(PACK_GRID,),
    in_specs=[
        pl.BlockSpec((DIM, PACK_COLS),
                     lambda i, q=q: (0, jnp.minimum(i + q * PACK_GRID,
                                                    DICT // PACK_COLS)))
        for q in range(4)
    ],
    out_specs=pl.BlockSpec((PACK_COLS, 2 * DIM), lambda i: (i, 0)),
)


@functools.partial(
    pl.kernel,
    out_type=jax.ShapeDtypeStruct((BATCH, DIM), jnp.float32),
    mesh=_mesh,
    scratch_types=[
        pltpu.VMEM((NUM_CHUNKS, CHUNK_ROWS), jnp.int32),
        pltpu.VMEM((ITEMS_PER_WORKER, DIM), jnp.int32),
        pltpu.VMEM((NBUF, CHUNK_ROWS, 2 * DIM), jnp.int32),
        pltpu.VMEM((ITEMS_PER_WORKER, DIM), jnp.float32),
        pltpu.SemaphoreType.DMA,
        pltpu.SemaphoreType.DMA,
    ],
)
def _bow_pool_sc(pair_hbm, par_hbm, table_hbm, out_hbm,
                 idx_v, par_v, rows_v, bow_v, sem0, sem1):
    wid = lax.axis_index("s") * NUM_CORES + lax.axis_index("c")
    sems = [sem0, sem1]

    pltpu.sync_copy(pair_hbm.at[pl.ds(wid * NUM_CHUNKS, NUM_CHUNKS)], idx_v)
    pltpu.sync_copy(
        par_hbm.at[pl.ds(wid * ITEMS_PER_WORKER, ITEMS_PER_WORKER)], par_v)

    def start_gather(g, slot):
        pltpu.async_copy(table_hbm.at[idx_v.at[g]], rows_v.at[slot], sems[slot])

    for slot in range(NBUF):
        start_gather(slot, slot)

    def step(i, carry):
        for slot in range(NBUF):
            g = i * NBUF + slot
            pltpu.make_async_copy(
                table_hbm.at[idx_v.at[g]], rows_v.at[slot], sems[slot]).wait()
            for item in range(CHUNK_ITEMS):
                base = item * SEQ
                pvs = [par_v[g * CHUNK_ITEMS + item, pl.ds(s, LANES)]
                       for s in (0, 16, 32, 48)]
                def par_of(r):
                    return pvs[r // 16][r % 16]
                def halves(r, off):
                    off = pl.multiple_of(off, 2 * LANES)
                    los, his = [], []
                    for h in range(2):
                        w32 = rows_v[slot, base + r,
                                     pl.ds(off + h * LANES, LANES)]
                        u = w32
                        los.append(jax.lax.bitcast_convert_type(
                            u << 16, jnp.float32))
                        his.append(jax.lax.bitcast_convert_type(
                            u & jnp.int32(-65536), jnp.float32))
                    return los + his

                accs = halves(0, par_of(0))
                for r in range(1, SEQ):
                    vs = halves(r, par_of(r))
                    for d in range(DIM // LANES):
                        accs[d] = accs[d] + vs[d]
                row_out = g * CHUNK_ITEMS + item
                for d in range(DIM // LANES):
                    bow_v[row_out, pl.ds(d * LANES, LANES)] = accs[d]
            @pl.when(i < NSTEPS - 1)
            def _():
                start_gather(g + NBUF, slot)
        return carry

    lax.fori_loop(0, NSTEPS, step, 0)
    pltpu.sync_copy(
        bow_v, out_hbm.at[pl.ds(wid * ITEMS_PER_WORKER, ITEMS_PER_WORKER)])


def _hidden_tc(x_ref, w_ref, b_ref, o_ref):
    acc = jax.lax.dot_general(
        x_ref[...], w_ref[...], (((1,), (0,)), ((), ())),
        preferred_element_type=jnp.float32)
    o_ref[...] = jnp.maximum(acc + b_ref[...], 0.0)


_hidden_call = pl.pallas_call(
    _hidden_tc,
    out_shape=jax.ShapeDtypeStruct((BATCH, DIM), jnp.float32),
)

def kernel(sentence, table, W, b):
    sent = sentence.astype(jnp.int32)
    pair = (sent & (PACK_SPLIT - 1)).reshape(BATCH * SEQ // CHUNK_ROWS,
                                             CHUNK_ROWS)
    par = jnp.pad((sent >> 18) << 5, ((0, 0), (0, DIM - SEQ)))
    tt = table.T
    table2 = _pack_call(tt, tt, tt, tt)
    bow = _bow_pool_sc(pair, par, table2)
    return _hidden_call(bow, W.T, b.reshape(1, DIM))


# speedup vs baseline: 1.4643x; 1.4643x over previous
import functools

import jax
import jax.numpy as jnp
from jax import lax
from jax.experimental import pallas as pl
from jax.experimental.pallas import tpu as pltpu
from jax.experimental.pallas import tpu_sc as plsc

DICT = 1000000
DICT_HALF = DICT // 2
BATCH = 4096
SEQ = 50
DIM = 64
LANES = 16

NUM_CORES = 2
NUM_SUBCORES = 16
NUM_WORKERS = NUM_CORES * NUM_SUBCORES

ITEMS_PER_WORKER = BATCH // NUM_WORKERS
CHUNK_ITEMS = 2
CHUNK_ROWS = CHUNK_ITEMS * SEQ
NUM_CHUNKS = ITEMS_PER_WORKER // CHUNK_ITEMS
NBUF = 2
NSTEPS = NUM_CHUNKS // NBUF

PACK_SPLIT = 524288
PACK_COLS = 8192
PACK_SUB = 512
PACK_GRID = PACK_SPLIT // PACK_COLS

_mesh = plsc.VectorSubcoreMesh(
    core_axis_name="c", subcore_axis_name="s",
    num_cores=NUM_CORES, num_subcores=NUM_SUBCORES)


def _pack_tc(a_ref, b_ref, o_ref):
    for k in range(PACK_COLS // PACK_SUB):
        s = k * PACK_SUB
        o_ref[pl.ds(s, PACK_SUB), :DIM] = jnp.transpose(
            a_ref[:, pl.ds(s, PACK_SUB)])
        o_ref[pl.ds(s, PACK_SUB), DIM:] = jnp.transpose(
            b_ref[:, pl.ds(s, PACK_SUB)])


_pack_call = pl.pallas_call(
    _pack_tc,
    out_shape=jax.ShapeDtypeStruct((PACK_SPLIT, 2 * DIM), jnp.float32),
    grid=(PACK_GRID,),
    in_specs=[
        pl.BlockSpec((DIM, PACK_COLS), lambda i: (0, i)),
        pl.BlockSpec((DIM, PACK_COLS),
                     lambda i: (0, jnp.minimum(i + PACK_GRID, DICT // PACK_COLS))),
    ],
    out_specs=pl.BlockSpec((PACK_COLS, 2 * DIM), lambda i: (i, 0)),
)


@functools.partial(
    pl.kernel,
    out_type=jax.ShapeDtypeStruct((BATCH, DIM), jnp.float32),
    mesh=_mesh,
    scratch_types=[
        pltpu.VMEM((NUM_CHUNKS, CHUNK_ROWS), jnp.int32),
        pltpu.VMEM((NUM_CHUNKS * CHUNK_ROWS,), jnp.int32),
        pltpu.VMEM((NBUF, CHUNK_ROWS, 2 * DIM), jnp.float32),
        pltpu.VMEM((ITEMS_PER_WORKER, DIM), jnp.float32),
        pltpu.SemaphoreType.DMA,
        pltpu.SemaphoreType.DMA,
    ],
)
def _bow_pool_sc(pair_hbm, par_hbm, table_hbm, out_hbm,
                 idx_v, par_v, rows_v, bow_v, sem0, sem1):
    wid = lax.axis_index("s") * NUM_CORES + lax.axis_index("c")
    sems = [sem0, sem1]

    pltpu.sync_copy(pair_hbm.at[pl.ds(wid * NUM_CHUNKS, NUM_CHUNKS)], idx_v)
    pltpu.sync_copy(
        par_hbm.at[pl.ds(wid * NUM_CHUNKS * CHUNK_ROWS, NUM_CHUNKS * CHUNK_ROWS)],
        par_v)

    def start_gather(g, slot):
        pltpu.async_copy(table_hbm.at[idx_v.at[g]], rows_v.at[slot], sems[slot])

    for slot in range(NBUF):
        start_gather(slot, slot)

    def step(i, carry):
        for slot in range(NBUF):
            g = i * NBUF + slot
            pltpu.make_async_copy(
                table_hbm.at[idx_v.at[g]], rows_v.at[slot], sems[slot]).wait()
            for item in range(CHUNK_ITEMS):
                base = item * SEQ
                po = (g * CHUNK_ITEMS + item) * SEQ
                pvs = [par_v[pl.ds(po + s, LANES)] for s in (0, 16, 32, 34)]
                def par_of(r):
                    if r < 48:
                        return pvs[r // 16][r % 16]
                    return pvs[3][r - 34]
                off0 = par_of(0)
                accs = [rows_v[slot, base, pl.ds(off0 + d * LANES, LANES)]
                        for d in range(DIM // LANES)]
                for r in range(1, SEQ):
                    off = par_of(r)
                    for d in range(DIM // LANES):
                        accs[d] = accs[d] + rows_v[slot, base + r,
                                                   pl.ds(off + d * LANES, LANES)]
                row_out = g * CHUNK_ITEMS + item
                for d in range(DIM // LANES):
                    bow_v[row_out, pl.ds(d * LANES, LANES)] = accs[d]
            @pl.when(i < NSTEPS - 1)
            def _():
                start_gather(g + NBUF, slot)
        return carry

    lax.fori_loop(0, NSTEPS, step, 0)
    pltpu.sync_copy(
        bow_v, out_hbm.at[pl.ds(wid * ITEMS_PER_WORKER, ITEMS_PER_WORKER)])


def _hidden_tc(x_ref, w_ref, b_ref, o_ref):
    acc = jax.lax.dot_general(
        x_ref[...], w_ref[...], (((1,), (0,)), ((), ())),
        preferred_element_type=jnp.float32)
    o_ref[...] = jnp.maximum(acc + b_ref[...], 0.0)


_hidden_call = pl.pallas_call(
    _hidden_tc,
    out_shape=jax.ShapeDtypeStruct((BATCH, DIM), jnp.float32),
)


def kernel(sentence, table, W, b):
    sent = sentence.astype(jnp.int32)
    in_hi = sent >= PACK_SPLIT
    pair = jnp.where(in_hi, sent - PACK_SPLIT, sent)
    pair = pair.reshape(BATCH * SEQ // CHUNK_ROWS, CHUNK_ROWS)
    par = jnp.where(in_hi, DIM, 0).reshape(BATCH * SEQ)
    tt = table.T
    table2 = _pack_call(tt, tt)
    bow = _bow_pool_sc(pair, par, table2)
    return _hidden_call(bow, W.T, b.reshape(1, DIM))


# speedup vs baseline: 1.5347x; 1.0481x over previous
import functools

import jax
import jax.numpy as jnp
from jax import lax
from jax.experimental import pallas as pl
from jax.experimental.pallas import tpu as pltpu
from jax.experimental.pallas import tpu_sc as plsc

DICT = 1000000
DICT_HALF = DICT // 2
BATCH = 4096
SEQ = 50
DIM = 64
LANES = 16

NUM_CORES = 2
NUM_SUBCORES = 16
NUM_WORKERS = NUM_CORES * NUM_SUBCORES

ITEMS_PER_WORKER = BATCH // NUM_WORKERS
CHUNK_ITEMS = 2
CHUNK_ROWS = CHUNK_ITEMS * SEQ
NUM_CHUNKS = ITEMS_PER_WORKER // CHUNK_ITEMS
NBUF = 2
NSTEPS = NUM_CHUNKS // NBUF

PACK_SPLIT = 524288
PACK_COLS = 16384
PACK_SUB = 512
PACK_GRID = PACK_SPLIT // PACK_COLS

_mesh = plsc.VectorSubcoreMesh(
    core_axis_name="c", subcore_axis_name="s",
    num_cores=NUM_CORES, num_subcores=NUM_SUBCORES)


def _pack_tc(a_ref, b_ref, o_ref):
    for k in range(PACK_COLS // PACK_SUB):
        s = k * PACK_SUB
        o_ref[pl.ds(s, PACK_SUB), :DIM] = jnp.transpose(
            a_ref[:, pl.ds(s, PACK_SUB)])
        o_ref[pl.ds(s, PACK_SUB), DIM:] = jnp.transpose(
            b_ref[:, pl.ds(s, PACK_SUB)])


_pack_call = pl.pallas_call(
    _pack_tc,
    out_shape=jax.ShapeDtypeStruct((PACK_SPLIT, 2 * DIM), jnp.float32),
    grid=(PACK_GRID,),
    in_specs=[
        pl.BlockSpec((DIM, PACK_COLS), lambda i: (0, i)),
        pl.BlockSpec((DIM, PACK_COLS),
                     lambda i: (0, jnp.minimum(i + PACK_GRID, DICT // PACK_COLS))),
    ],
    out_specs=pl.BlockSpec((PACK_COLS, 2 * DIM), lambda i: (i, 0)),
)


@functools.partial(
    pl.kernel,
    out_type=jax.ShapeDtypeStruct((BATCH, DIM), jnp.float32),
    mesh=_mesh,
    scratch_types=[
        pltpu.VMEM((NUM_CHUNKS, CHUNK_ROWS), jnp.int32),
        pltpu.VMEM((NUM_CHUNKS * CHUNK_ROWS,), jnp.int32),
        pltpu.VMEM((NBUF, CHUNK_ROWS, 2 * DIM), jnp.float32),
        pltpu.VMEM((ITEMS_PER_WORKER, DIM), jnp.float32),
        pltpu.SemaphoreType.DMA,
        pltpu.SemaphoreType.DMA,
    ],
)
def _bow_pool_sc(pair_hbm, par_hbm, table_hbm, out_hbm,
                 idx_v, par_v, rows_v, bow_v, sem0, sem1):
    wid = lax.axis_index("s") * NUM_CORES + lax.axis_index("c")
    sems = [sem0, sem1]

    pltpu.sync_copy(pair_hbm.at[pl.ds(wid * NUM_CHUNKS, NUM_CHUNKS)], idx_v)
    pltpu.sync_copy(
        par_hbm.at[pl.ds(wid * NUM_CHUNKS * CHUNK_ROWS, NUM_CHUNKS * CHUNK_ROWS)],
        par_v)

    def start_gather(g, slot):
        pltpu.async_copy(table_hbm.at[idx_v.at[g]], rows_v.at[slot], sems[slot])

    for slot in range(NBUF):
        start_gather(slot, slot)

    def step(i, carry):
        for slot in range(NBUF):
            g = i * NBUF + slot
            pltpu.make_async_copy(
                table_hbm.at[idx_v.at[g]], rows_v.at[slot], sems[slot]).wait()
            for item in range(CHUNK_ITEMS):
                base = item * SEQ
                po = (g * CHUNK_ITEMS + item) * SEQ
                pvs = [par_v[pl.ds(po + s, LANES)] for s in (0, 16, 32, 34)]
                def par_of(r):
                    if r < 48:
                        return pvs[r // 16][r % 16]
                    return pvs[3][r - 34]
                off0 = par_of(0)
                accs = [rows_v[slot, base, pl.ds(off0 + d * LANES, LANES)]
                        for d in range(DIM // LANES)]
                for r in range(1, SEQ):
                    off = par_of(r)
                    for d in range(DIM // LANES):
                        accs[d] = accs[d] + rows_v[slot, base + r,
                                                   pl.ds(off + d * LANES, LANES)]
                row_out = g * CHUNK_ITEMS + item
                for d in range(DIM // LANES):
                    bow_v[row_out, pl.ds(d * LANES, LANES)] = accs[d]
            @pl.when(i < NSTEPS - 1)
            def _():
                start_gather(g + NBUF, slot)
        return carry

    lax.fori_loop(0, NSTEPS, step, 0)
    pltpu.sync_copy(
        bow_v, out_hbm.at[pl.ds(wid * ITEMS_PER_WORKER, ITEMS_PER_WORKER)])


def _hidden_tc(x_ref, w_ref, b_ref, o_ref):
    acc = jax.lax.dot_general(
        x_ref[...], w_ref[...], (((1,), (0,)), ((), ())),
        preferred_element_type=jnp.float32)
    o_ref[...] = jnp.maximum(acc + b_ref[...], 0.0)


_hidden_call = pl.pallas_call(
    _hidden_tc,
    out_shape=jax.ShapeDtypeStruct((BATCH, DIM), jnp.float32),
)


def kernel(sentence, table, W, b):
    sent = sentence.astype(jnp.int32)
    in_hi = sent >= PACK_SPLIT
    pair = jnp.where(in_hi, sent - PACK_SPLIT, sent)
    pair = pair.reshape(BATCH * SEQ // CHUNK_ROWS, CHUNK_ROWS)
    par = jnp.where(in_hi, DIM, 0).reshape(BATCH * SEQ)
    tt = table.T
    table2 = _pack_call(tt, tt)
    bow = _bow_pool_sc(pair, par, table2)
    return _hidden_call(bow, W.T, b.reshape(1, DIM))
